# baseline (speedup 1.0000x reference)
import jax
import jax.numpy as jnp
from jax import lax
from jax.experimental import pallas as pl
from jax.experimental.pallas import tpu as pltpu
from jax.experimental.pallas import tpu_sc as plsc

MARGIN = 1.0
CH = 128
LANES = 16


def _sc_distance_body(h_idx, r_idx, t_idx, ent, rel, out,
                      idxh_v, idxr_v, idxt_v, rows_h, rows_r, rows_t,
                      dist_v, sem):
    dim = ent.shape[1]
    total = h_idx.shape[0]
    nc = lax.axis_size("c")
    wid = lax.axis_index("s") * nc + lax.axis_index("c")
    per_w = total // (nc * lax.axis_size("s"))
    n_chunks = per_w // CH
    w_base = wid * per_w

    row_ids = lax.iota(jnp.int32, LANES)

    def chunk_body(c, carry):
        base = pl.multiple_of(w_base + c * CH, CH)
        pltpu.sync_copy(h_idx.at[pl.ds(base, CH)], idxh_v)
        pltpu.sync_copy(r_idx.at[pl.ds(base, CH)], idxr_v)
        pltpu.sync_copy(t_idx.at[pl.ds(base, CH)], idxt_v)
        cp_h = pltpu.async_copy(ent.at[idxh_v], rows_h, sem)
        cp_r = pltpu.async_copy(rel.at[idxr_v], rows_r, sem)
        cp_t = pltpu.async_copy(ent.at[idxt_v], rows_t, sem)
        cp_h.wait()
        cp_r.wait()
        cp_t.wait()
        for g in range(CH // LANES):
            rows16 = row_ids + (g * LANES)

            def dim_body(d4, acc):
                for k in range(4):
                    col = jnp.full((LANES,), d4 * 4 + k, jnp.int32)
                    h = plsc.load_gather(rows_h, [rows16, col])
                    r = plsc.load_gather(rows_r, [rows16, col])
                    t = plsc.load_gather(rows_t, [rows16, col])
                    acc = acc + jnp.abs(h + r - t)
                return acc

            acc = lax.fori_loop(0, dim // 4, dim_body,
                                jnp.zeros((LANES,), jnp.float32))
            dist_v[pl.ds(g * LANES, LANES)] = acc
        pltpu.sync_copy(dist_v, out.at[pl.ds(base, CH)])
        return carry

    lax.fori_loop(0, n_chunks, chunk_body, 0)


def _loss_body(d_ref, loss_ref):
    d = d_ref[...]
    pos = d[:, 0:1]
    neg = d[:, 1:]
    hinge = jnp.maximum(pos - neg + MARGIN, 0.0)
    loss_ref[0, 0] = jnp.sum(hinge) / (neg.shape[0] * neg.shape[1])


@jax.jit
def kernel(triple_matrix, entities_emb, relations_emb):
    b, np1, _ = triple_matrix.shape
    total = b * np1
    flat = triple_matrix.reshape(total, 3)
    h_idx = flat[:, 0]
    r_idx = flat[:, 1]
    t_idx = flat[:, 2]

    mesh = plsc.VectorSubcoreMesh(core_axis_name="c", subcore_axis_name="s")
    dim = entities_emb.shape[1]
    dist_flat = pl.kernel(
        _sc_distance_body,
        out_type=jax.ShapeDtypeStruct((total,), jnp.float32),
        mesh=mesh,
        compiler_params=pltpu.CompilerParams(
            needs_layout_passes=False, use_tc_tiling_on_sc=False),
        scratch_types=[
            pltpu.VMEM((CH,), jnp.int32),
            pltpu.VMEM((CH,), jnp.int32),
            pltpu.VMEM((CH,), jnp.int32),
            pltpu.VMEM((CH, dim), jnp.float32),
            pltpu.VMEM((CH, dim), jnp.float32),
            pltpu.VMEM((CH, dim), jnp.float32),
            pltpu.VMEM((CH,), jnp.float32),
            pltpu.SemaphoreType.DMA,
        ],
    )(h_idx, r_idx, t_idx, entities_emb, relations_emb)

    d2 = dist_flat.reshape(b, np1)
    pos = d2[:, 0]
    neg = d2[:, 1:]

    loss = pl.pallas_call(
        _loss_body,
        out_shape=jax.ShapeDtypeStruct((1, 1), jnp.float32),
        out_specs=pl.BlockSpec(memory_space=pltpu.SMEM),
    )(d2)[0, 0]

    return (loss, pos, neg)


# speedup vs baseline: 1.1002x; 1.1002x over previous
import jax
import jax.numpy as jnp
from jax import lax
from jax.experimental import pallas as pl
from jax.experimental.pallas import tpu as pltpu
from jax.experimental.pallas import tpu_sc as plsc

MARGIN = 1.0
LANES = 16
CH = 208
SUB = 104
NSUB = CH // SUB


def _sc_distance_body(h_idx, r_idx, t_idx, ent, rel, out,
                      idxh_v, idxr_v, idxt_v, dist_v,
                      h0, r0, t0, h1, r1, t1,
                      sem_idx, sem_g0, sem_g1):
    dim = ent.shape[1]
    total = h_idx.shape[0]
    nc = lax.axis_size("c")
    nw = nc * lax.axis_size("s")
    wid = lax.axis_index("s") * nc + lax.axis_index("c")
    rpw = total // nw
    nch = rpw // CH
    pairs = nch // 2
    w_base = wid * rpw

    row_ids = lax.iota(jnp.int32, LANES)

    cps = [pltpu.async_copy(src.at[pl.ds(w_base, rpw)], dst, sem_idx)
           for src, dst in ((h_idx, idxh_v), (r_idx, idxr_v), (t_idx, idxt_v))]
    for cp in cps:
        cp.wait()

    def fire(c, bufs, sem):
        hB, rB, tB = bufs
        for j in range(NSUB):
            src_sl = pl.ds(c * CH + j * SUB, SUB)
            dst_sl = pl.ds(j * SUB, SUB)
            pltpu.async_copy(ent.at[idxh_v.at[src_sl]], hB.at[dst_sl], sem)
            pltpu.async_copy(rel.at[idxr_v.at[src_sl]], rB.at[dst_sl], sem)
            pltpu.async_copy(ent.at[idxt_v.at[src_sl]], tB.at[dst_sl], sem)

    def drain(bufs, sem):
        hB, rB, tB = bufs
        for j in range(NSUB):
            sl = pl.ds(j * SUB, SUB)
            dum = pl.ds(0, SUB)
            pltpu.make_async_copy(ent.at[idxh_v.at[dum]], hB.at[sl], sem).wait()
            pltpu.make_async_copy(rel.at[idxr_v.at[dum]], rB.at[sl], sem).wait()
            pltpu.make_async_copy(ent.at[idxt_v.at[dum]], tB.at[sl], sem).wait()

    def compute(c, bufs):
        hB, rB, tB = bufs
        for g in range(CH // LANES):
            rows16 = row_ids + (g * LANES)

            def dim_body(d4, acc):
                for k in range(4):
                    col = jnp.full((LANES,), d4 * 4 + k, jnp.int32)
                    h = plsc.load_gather(hB, [rows16, col])
                    r = plsc.load_gather(rB, [rows16, col])
                    t = plsc.load_gather(tB, [rows16, col])
                    acc = acc + jnp.abs(h + r - t)
                return acc

            acc = lax.fori_loop(0, dim // 4, dim_body,
                                jnp.zeros((LANES,), jnp.float32))
            dist_v[pl.ds(c * CH + g * LANES, LANES)] = acc

    bufs0 = (h0, r0, t0)
    bufs1 = (h1, r1, t1)

    fire(0, bufs0, sem_g0)

    def pair_body(p, carry):
        c0 = p * 2
        fire(c0 + 1, bufs1, sem_g1)
        drain(bufs0, sem_g0)
        compute(c0, bufs0)

        @pl.when(p + 1 < pairs)
        def _():
            fire(c0 + 2, bufs0, sem_g0)

        drain(bufs1, sem_g1)
        compute(c0 + 1, bufs1)
        return carry

    lax.fori_loop(0, pairs, pair_body, 0)

    pltpu.sync_copy(dist_v, out.at[pl.ds(w_base, rpw)])


def _loss_body(d_ref, loss_ref):
    d = d_ref[...]
    pos = d[:, 0:1]
    neg = d[:, 1:]
    hinge = jnp.maximum(pos - neg + MARGIN, 0.0)
    loss_ref[0, 0] = jnp.sum(hinge) / (neg.shape[0] * neg.shape[1])


@jax.jit
def kernel(triple_matrix, entities_emb, relations_emb):
    b, np1, _ = triple_matrix.shape
    total = b * np1
    flat = triple_matrix.reshape(total, 3)
    h_idx = flat[:, 0]
    r_idx = flat[:, 1]
    t_idx = flat[:, 2]

    mesh = plsc.VectorSubcoreMesh(core_axis_name="c", subcore_axis_name="s")
    dim = entities_emb.shape[1]
    rpw = total // 32
    dist_flat = pl.kernel(
        _sc_distance_body,
        out_type=jax.ShapeDtypeStruct((total,), jnp.float32),
        mesh=mesh,
        compiler_params=pltpu.CompilerParams(
            needs_layout_passes=False, use_tc_tiling_on_sc=False),
        scratch_types=[
            pltpu.VMEM((rpw,), jnp.int32),
            pltpu.VMEM((rpw,), jnp.int32),
            pltpu.VMEM((rpw,), jnp.int32),
            pltpu.VMEM((rpw,), jnp.float32),
            pltpu.VMEM((CH, dim), jnp.float32),
            pltpu.VMEM((CH, dim), jnp.float32),
            pltpu.VMEM((CH, dim), jnp.float32),
            pltpu.VMEM((CH, dim), jnp.float32),
            pltpu.VMEM((CH, dim), jnp.float32),
            pltpu.VMEM((CH, dim), jnp.float32),
            pltpu.SemaphoreType.DMA,
            pltpu.SemaphoreType.DMA,
            pltpu.SemaphoreType.DMA,
        ],
    )(h_idx, r_idx, t_idx, entities_emb, relations_emb)

    d2 = dist_flat.reshape(b, np1)
    pos = d2[:, 0]
    neg = d2[:, 1:]

    loss = pl.pallas_call(
        _loss_body,
        out_shape=jax.ShapeDtypeStruct((1, 1), jnp.float32),
        out_specs=pl.BlockSpec(memory_space=pltpu.SMEM),
    )(d2)[0, 0]

    return (loss, pos, neg)
